# baseline (speedup 1.0000x reference)
import jax
import jax.numpy as jnp
from jax.experimental import pallas as pl

D_EDGE = 16
NUM_HEADS = 4


def _copy_body(x_ref, o_ref):
    o_ref[...] = x_ref[...]


def kernel(node_features, edge_features, edge_index, node_tiers,
           w_ns, b_ns, w_es, b_es, Wnq, Wk, Wv, Wo, W1, b1, W2, b2):
    N = node_features.shape[0]
    E = edge_features.shape[0]
    H = NUM_HEADS
    dh = D_EDGE // H
    ns = (node_features @ w_ns + b_ns)[:, 0]
    es = (edge_features @ w_es + b_es)[:, 0]
    k_n = N // 2
    k_e = E // 2
    node_thr = jax.lax.top_k(ns, k_n)[0][-1]
    edge_thr = jax.lax.top_k(es, k_e)[0][-1]
    src = edge_index[0]
    dst = edge_index[1]
    node_mask = ns >= node_thr
    edge_mask = (es >= edge_thr) & node_mask[src] & node_mask[dst]
    ef = edge_features * edge_mask[:, None].astype(edge_features.dtype)
    qn = (node_features @ Wnq).reshape(N, H, dh)
    ke = (ef @ Wk).reshape(E, H, dh)
    ve = (ef @ Wv).reshape(E, H, dh)
    logits = jnp.sum(qn[src] * ke, axis=-1) / jnp.sqrt(jnp.float32(dh))
    seg_max = jax.ops.segment_max(logits, src, num_segments=N)
    seg_max = jnp.where(jnp.isfinite(seg_max), seg_max, 0.0)
    ex = jnp.exp(logits - seg_max[src])
    denom = jax.ops.segment_sum(ex, src, num_segments=N)
    attn = ex / (denom[src] + 1e-9)
    msg = jax.ops.segment_sum(attn[..., None] * ve, src, num_segments=N)
    edge_out = ef + msg[src].reshape(E, H * dh) @ Wo
    h = jax.nn.gelu(edge_out @ W1 + b1)
    cls = h @ W2 + b2
    cls = pl.pallas_call(
        _copy_body,
        grid=(E // 2000,),
        in_specs=[pl.BlockSpec((2000, D_EDGE), lambda i: (i, 0))],
        out_specs=pl.BlockSpec((2000, D_EDGE), lambda i: (i, 0)),
        out_shape=jax.ShapeDtypeStruct(cls.shape, cls.dtype),
    )(cls)
    return (cls, jnp.float32(0.0))


# speedup vs baseline: 27.5507x; 27.5507x over previous
import functools

import jax
import jax.numpy as jnp
from jax import lax
from jax.experimental import pallas as pl
from jax.experimental.pallas import tpu as pltpu
from jax.experimental.pallas import tpu_sc as plsc

D_NODE = 128
D_EDGE = 16
N_CLS = 16
H = 4
DH = 4
N = 10000
E = 320000

NC = 2
NS = 16
NW = NC * NS
EPW = E // NW
CE = 400
NCH = EPW // CE
G = CE // 16
AW = 32
NP = 10240
RPT = NP // NS

EB = 2000
NEB = E // EB

CE2 = 2000
NCH2 = EPW // CE2


def _sortable_key(x):
    b = lax.bitcast_convert_type(x, jnp.int32)
    return b ^ jnp.bitwise_and(lax.shift_right_arithmetic(b, 31), jnp.int32(0x7FFFFFFF))


def _a1_body(ef_ref, wes_ref, bes_ref, wk_ref, wv_ref, es_ref, k_ref, v_ref):
    ef = ef_ref[...]
    es_ref[...] = jnp.dot(ef, wes_ref[...], preferred_element_type=jnp.float32) + bes_ref[0, 0]
    k_ref[...] = jnp.dot(ef, wk_ref[...], preferred_element_type=jnp.float32)
    v_ref[...] = jnp.dot(ef, wv_ref[...], preferred_element_type=jnp.float32)


def _run_a1(edge_features, w_es, b_es2, Wk, Wv):
    return pl.pallas_call(
        _a1_body,
        grid=(NEB,),
        in_specs=[
            pl.BlockSpec((EB, D_EDGE), lambda i: (i, 0)),
            pl.BlockSpec((D_EDGE, 1), lambda i: (0, 0)),
            pl.BlockSpec((1, 1), lambda i: (0, 0)),
            pl.BlockSpec((D_EDGE, D_EDGE), lambda i: (0, 0)),
            pl.BlockSpec((D_EDGE, D_EDGE), lambda i: (0, 0)),
        ],
        out_specs=[
            pl.BlockSpec((EB, 1), lambda i: (i, 0)),
            pl.BlockSpec((EB, D_EDGE), lambda i: (i, 0)),
            pl.BlockSpec((EB, D_EDGE), lambda i: (i, 0)),
        ],
        out_shape=[
            jax.ShapeDtypeStruct((E, 1), jnp.float32),
            jax.ShapeDtypeStruct((E, D_EDGE), jnp.float32),
            jax.ShapeDtypeStruct((E, D_EDGE), jnp.float32),
        ],
    )(edge_features, w_es, b_es2, Wk, Wv)


def _a2_body(nf_ref, wns_ref, bns_ref, wnq_ref, es_ref, qn_ref, nm_ref, thr_ref):
    nf = nf_ref[...]
    ns = jnp.dot(nf, wns_ref[...], preferred_element_type=jnp.float32) + bns_ref[0, 0]
    qn_ref[...] = jnp.dot(nf, wnq_ref[...], preferred_element_type=jnp.float32)
    nkey = _sortable_key(ns)
    ekey = _sortable_key(es_ref[...])
    int_min = jnp.int32(-2147483648)

    def bit_body(i, carry):
        offn, offe = carry
        bit = lax.shift_left(jnp.int32(1), 31 - i)
        trn = offn + bit
        tre = offe + bit
        cn = jnp.sum((nkey >= int_min + trn).astype(jnp.int32))
        ce = jnp.sum((ekey >= int_min + tre).astype(jnp.int32))
        offn = jnp.where(cn >= N // 2, trn, offn)
        offe = jnp.where(ce >= E // 2, tre, offe)
        return offn, offe

    offn, offe = lax.fori_loop(0, 32, bit_body, (jnp.int32(0), jnp.int32(0)))
    tn = int_min + offn
    te = int_min + offe
    nm_ref[...] = (nkey >= tn).astype(jnp.float32)
    eb = jnp.where(te >= 0, te, te ^ jnp.int32(0x7FFFFFFF))
    thr_ref[...] = jnp.full((1, 16), lax.bitcast_convert_type(eb, jnp.float32))


def _run_a2(node_features, w_ns, b_ns2, Wnq, es2d):
    return pl.pallas_call(
        _a2_body,
        out_shape=[
            jax.ShapeDtypeStruct((N, D_EDGE), jnp.float32),
            jax.ShapeDtypeStruct((N, 1), jnp.float32),
            jax.ShapeDtypeStruct((1, 16), jnp.float32),
        ],
    )(node_features, w_ns, b_ns2, Wnq, es2d)


def _s1_body(qn_hbm, nm_hbm, thr_hbm, srce_hbm, dste_hbm, es_hbm, k_hbm, v_hbm,
             mask_hbm, apart_hbm,
             src_v, dst_v, es_v, kbuf, vbuf, qbuf, nm_v, thr_v, stage,
             mask_v, acc):
    cid = lax.axis_index("c")
    sid = lax.axis_index("s")
    wid = sid * NC + cid
    zero16 = jnp.zeros((16,), jnp.float32)

    def _zero_stage(i, c):
        stage[i, pl.ds(0, 16)] = zero16
        stage[i, pl.ds(16, 16)] = zero16
        return c

    lax.fori_loop(0, CE, _zero_stage, 0)

    pltpu.sync_copy(stage, acc.at[pl.ds(sid * RPT, CE)])
    pltpu.sync_copy(stage.at[pl.ds(0, RPT - CE)],
                    acc.at[pl.ds(sid * RPT + CE, RPT - CE)])
    pltpu.sync_copy(nm_hbm, nm_v)
    pltpu.sync_copy(thr_hbm, thr_v)
    plsc.subcore_barrier()

    thrv = thr_v[...]
    lane = lax.iota(jnp.int32, 16)

    def chunk_body(ci, c):
        base = wid * EPW + ci * CE
        pltpu.sync_copy(srce_hbm.at[pl.ds(base, CE)], src_v)
        pltpu.sync_copy(dste_hbm.at[pl.ds(base, CE)], dst_v)
        pltpu.sync_copy(es_hbm.at[pl.ds(base, CE)], es_v)
        pltpu.sync_copy(k_hbm.at[pl.ds(base, CE)], kbuf)
        pltpu.sync_copy(v_hbm.at[pl.ds(base, CE)], vbuf)
        pltpu.sync_copy(qn_hbm.at[src_v], qbuf)

        for g in range(G):
            sl = pl.ds(g * 16, 16)
            srcg = src_v[sl]
            dstg = dst_v[sl]
            esg = es_v[sl]
            nms = plsc.load_gather(nm_v, [srcg])
            nmd = plsc.load_gather(nm_v, [dstg])
            emask = jnp.where(esg >= thrv, 1.0, 0.0) * nms * nmd
            mask_v[sl] = emask
            halfm = emask * 0.5
            rowi = lane + (g * 16)
            exs = []
            for h in range(H):
                acc_l = None
                for d in range(DH):
                    hd = h * DH + d
                    colv = jnp.full((16,), hd, jnp.int32)
                    qv = plsc.load_gather(qbuf, [rowi, colv])
                    kv = plsc.load_gather(kbuf, [rowi, colv])
                    t = qv * kv
                    acc_l = t if acc_l is None else acc_l + t
                exh = jnp.exp(acc_l * halfm)
                exs.append(exh)
                plsc.store_scatter(
                    stage, [rowi, jnp.full((16,), 16 + h, jnp.int32)], exh)
            for h in range(H):
                exm = exs[h] * emask
                for d in range(DH):
                    hd = h * DH + d
                    colv = jnp.full((16,), hd, jnp.int32)
                    vv = plsc.load_gather(vbuf, [rowi, colv])
                    plsc.store_scatter(stage, [rowi, colv], exm * vv)

        pltpu.sync_copy(stage, acc.at[src_v], add=True)
        pltpu.sync_copy(mask_v, mask_hbm.at[pl.ds(base, CE)])
        return c

    lax.fori_loop(0, NCH, chunk_body, 0)
    plsc.subcore_barrier()
    pltpu.sync_copy(acc.at[pl.ds(sid * RPT, RPT)],
                    apart_hbm.at[cid, pl.ds(sid * RPT, RPT)])


def _run_s1(qn, nm, thrv, srce, dste, es2d, K, V):
    mesh = plsc.VectorSubcoreMesh(
        core_axis_name="c", subcore_axis_name="s",
        num_cores=NC, num_subcores=NS)
    f = pl.kernel(
        _s1_body,
        compiler_params=pltpu.CompilerParams(needs_layout_passes=False, use_tc_tiling_on_sc=False),
        out_type=[
            jax.ShapeDtypeStruct((E,), jnp.float32),
            jax.ShapeDtypeStruct((NC, NP, AW), jnp.float32),
        ],
        mesh=mesh,
        scratch_types=[
            pltpu.VMEM((CE,), jnp.int32),
            pltpu.VMEM((CE,), jnp.int32),
            pltpu.VMEM((CE,), jnp.float32),
            pltpu.VMEM((CE, D_EDGE), jnp.float32),
            pltpu.VMEM((CE, D_EDGE), jnp.float32),
            pltpu.VMEM((CE, D_EDGE), jnp.float32),
            pltpu.VMEM((N,), jnp.float32),
            pltpu.VMEM((16,), jnp.float32),
            pltpu.VMEM((CE, AW), jnp.float32),
            pltpu.VMEM((CE,), jnp.float32),
            pltpu.VMEM_SHARED((NP, AW), jnp.float32),
        ],
    )
    return f(qn, nm, thrv, srce, dste, es2d, K, V)


def _b_body(ap_ref, wo_ref, w1_ref, b1_ref, y_ref):
    a = ap_ref[0] + ap_ref[1]
    m = a[:, :D_EDGE]
    msg = jnp.concatenate(
        [m[:, h * DH:(h + 1) * DH] / (a[:, D_EDGE + h:D_EDGE + h + 1] + 1e-9)
         for h in range(H)], axis=1)
    y = jnp.dot(msg, wo_ref[...], preferred_element_type=jnp.float32)
    y_ref[...] = jnp.dot(y, w1_ref[...], preferred_element_type=jnp.float32) + b1_ref[...]


def _run_b(apart, Wo, W1, b12):
    return pl.pallas_call(
        _b_body,
        out_shape=jax.ShapeDtypeStruct((NP, D_EDGE), jnp.float32),
    )(apart, Wo, W1, b12)


def _s2_body(srce_hbm, y_hbm, z_hbm, idx_v, rows_v):
    cid = lax.axis_index("c")
    sid = lax.axis_index("s")
    wid = sid * NC + cid

    def body(ci, c):
        base = wid * EPW + ci * CE2
        pltpu.sync_copy(srce_hbm.at[pl.ds(base, CE2)], idx_v)
        pltpu.sync_copy(y_hbm.at[idx_v], rows_v)
        pltpu.sync_copy(rows_v, z_hbm.at[pl.ds(base, CE2)])
        return c

    lax.fori_loop(0, NCH2, body, 0)


def _run_s2(srce, Y):
    mesh = plsc.VectorSubcoreMesh(
        core_axis_name="c", subcore_axis_name="s",
        num_cores=NC, num_subcores=NS)
    f = pl.kernel(
        _s2_body,
        compiler_params=pltpu.CompilerParams(needs_layout_passes=False, use_tc_tiling_on_sc=False),
        out_type=jax.ShapeDtypeStruct((E, D_EDGE), jnp.float32),
        mesh=mesh,
        scratch_types=[
            pltpu.VMEM((CE2,), jnp.int32),
            pltpu.VMEM((CE2, D_EDGE), jnp.float32),
        ],
    )
    return f(srce, Y)


def _c_body(ef_ref, mk_ref, z_ref, w1_ref, w2_ref, b2_ref, out_ref):
    ef = ef_ref[...]
    x1 = jnp.dot(ef, w1_ref[...], preferred_element_type=jnp.float32)
    pre = x1 * mk_ref[...] + z_ref[...]
    hh = jax.nn.gelu(pre)
    out_ref[...] = jnp.dot(hh, w2_ref[...], preferred_element_type=jnp.float32) + b2_ref[...]


def _run_c(edge_features, mask2d, Z, W1, W2, b22):
    return pl.pallas_call(
        _c_body,
        grid=(NEB,),
        in_specs=[
            pl.BlockSpec((EB, D_EDGE), lambda i: (i, 0)),
            pl.BlockSpec((EB, 1), lambda i: (i, 0)),
            pl.BlockSpec((EB, D_EDGE), lambda i: (i, 0)),
            pl.BlockSpec((D_EDGE, D_EDGE), lambda i: (0, 0)),
            pl.BlockSpec((D_EDGE, N_CLS), lambda i: (0, 0)),
            pl.BlockSpec((1, N_CLS), lambda i: (0, 0)),
        ],
        out_specs=pl.BlockSpec((EB, N_CLS), lambda i: (i, 0)),
        out_shape=jax.ShapeDtypeStruct((E, N_CLS), jnp.float32),
    )(edge_features, mask2d, Z, W1, W2, b22)


def kernel(node_features, edge_features, edge_index, node_tiers,
           w_ns, b_ns, w_es, b_es, Wnq, Wk, Wv, Wo, W1, b1, W2, b2):
    del node_tiers
    b_es2 = b_es.reshape(1, 1)
    b_ns2 = b_ns.reshape(1, 1)
    b12 = b1.reshape(1, D_EDGE)
    b22 = b2.reshape(1, N_CLS)

    es2d, K, V = _run_a1(edge_features, w_es, b_es2, Wk, Wv)
    qn, nm2d, thr2d = _run_a2(node_features, w_ns, b_ns2, Wnq,
                              es2d.reshape(NEB, EB))
    nm = nm2d.reshape(N)
    thrv = thr2d.reshape(16)
    srce = edge_index[0]
    dste = edge_index[1]
    es1d = es2d.reshape(E)
    qnp = jnp.pad(qn, ((0, NP - N), (0, 0)))
    maskE, apart = _run_s1(qnp, nm, thrv, srce, dste, es1d, K, V)
    Y = _run_b(apart, Wo, W1, b12)
    Z = _run_s2(srce, Y)
    cls = _run_c(edge_features, maskE.reshape(E, 1), Z, W1, W2, b22)
    return (cls, jnp.float32(0.0))
